# baseline (speedup 1.0000x reference)
import functools

import jax
import jax.numpy as jnp
from jax import lax
from jax.experimental import pallas as pl
from jax.experimental.pallas import tpu as pltpu
from jax.experimental.pallas import tpu_sc as plsc

B = 4096
S = 50
D = 64
V = 1000000
L = 16
NC = 2
NS = 16
NW = NC * NS
BW = B // NW
CB = 8
NCH = BW // CB
CHI = CB * S
PIECES = ((0, 128), (128, 128), (256, 128), (384, 16))

_MESH = plsc.VectorSubcoreMesh(
    core_axis_name="c", subcore_axis_name="s", num_cores=NC, num_subcores=NS
)


def _body(idx_hbm, table_hbm, out_hbm, idx_v, buf_v, outc_v, sem):
    wid = lax.axis_index("s") * NC + lax.axis_index("c")
    base = wid * (BW * S)
    pltpu.sync_copy(idx_hbm.at[pl.ds(base, BW * S)], idx_v)

    def chunk(c, carry):
        coff = c * CHI
        descs = []
        for off, n in PIECES:
            descs.append(
                pltpu.async_copy(
                    table_hbm.at[idx_v.at[pl.ds(coff + off, n)]],
                    buf_v.at[pl.ds(off, n)],
                    sem,
                )
            )
        for d in descs:
            d.wait()
        for r in range(CB):
            for dd in range(D // L):
                acc = buf_v[r * S, pl.ds(dd * L, L)]
                for j in range(1, S):
                    acc = acc + buf_v[r * S + j, pl.ds(dd * L, L)]
                outc_v[r, pl.ds(dd * L, L)] = acc * (1.0 / S)
        pltpu.sync_copy(outc_v, out_hbm.at[pl.ds(wid * BW + c * CB, CB)])
        return carry

    lax.fori_loop(0, NCH, chunk, 0)


@functools.partial(
    pl.kernel,
    out_type=jax.ShapeDtypeStruct((B, D), jnp.float32),
    mesh=_MESH,
    scratch_types=[
        pltpu.VMEM((BW * S,), jnp.int32),
        pltpu.VMEM((CHI, 2 * D), jnp.float32),
        pltpu.VMEM((CB, D), jnp.float32),
        pltpu.SemaphoreType.DMA,
    ],
    compiler_params=pltpu.CompilerParams(use_tc_tiling_on_sc=False),
)
def _embed_mean(idx_hbm, table_hbm, out_hbm, idx_v, buf_v, outc_v, sem):
    _body(idx_hbm, table_hbm, out_hbm, idx_v, buf_v, outc_v, sem)


TCW = 512
TCG = (V + TCW - 1) // TCW


def _tc_pad_transpose_body(tt_ref, out_ref):
    out_ref[:, :D] = tt_ref[...].T
    out_ref[:, D:] = jnp.zeros((TCW, D), jnp.float32)


_tc_pad_transpose = pl.pallas_call(
    _tc_pad_transpose_body,
    grid=(TCG,),
    in_specs=[pl.BlockSpec((D, TCW), lambda i: (0, i))],
    out_specs=pl.BlockSpec((TCW, 2 * D), lambda i: (i, 0)),
    out_shape=jax.ShapeDtypeStruct((V, 2 * D), jnp.float32),
)


def kernel(inputs, table):
    idx_flat = inputs.astype(jnp.int32).reshape(-1)
    tp = _tc_pad_transpose(table.T)
    return _embed_mean(idx_flat, tp)


# speedup vs baseline: 2.0276x; 2.0276x over previous
import functools

import jax
import jax.numpy as jnp
from jax import lax
from jax.experimental import pallas as pl
from jax.experimental.pallas import tpu as pltpu
from jax.experimental.pallas import tpu_sc as plsc

B = 4096
S = 50
D = 64
L = 16
NC = 2
NS = 16
NW = NC * NS
BW = B // NW
CB = 8
NCH = BW // CB
CHI = CB * S
PIECES = ((0, 128), (128, 128), (256, 128), (384, 16))

_MESH = plsc.VectorSubcoreMesh(
    core_axis_name="c", subcore_axis_name="s", num_cores=NC, num_subcores=NS
)


def _body(idx_hbm, table_hbm, out_hbm, idx_v, buf_v, outc_v, sem):
    wid = lax.axis_index("s") * NC + lax.axis_index("c")
    base = wid * (BW * S)
    pltpu.sync_copy(idx_hbm.at[pl.ds(base, BW * S)], idx_v)

    def chunk(c, carry):
        coff = c * CHI
        descs = []
        for off, n in PIECES:
            descs.append(
                pltpu.async_copy(
                    table_hbm.at[idx_v.at[pl.ds(coff + off, n)]],
                    buf_v.at[pl.ds(off, n)],
                    sem,
                )
            )
        for d in descs:
            d.wait()
        for r in range(CB):
            for dd in range(D // L):
                acc = buf_v[r * S, pl.ds(dd * L, L)]
                for j in range(1, S):
                    acc = acc + buf_v[r * S + j, pl.ds(dd * L, L)]
                outc_v[r, pl.ds(dd * L, L)] = acc * (1.0 / S)
        pltpu.sync_copy(outc_v, out_hbm.at[pl.ds(wid * BW + c * CB, CB)])
        return carry

    lax.fori_loop(0, NCH, chunk, 0)


@functools.partial(
    pl.kernel,
    out_type=jax.ShapeDtypeStruct((B, D), jnp.float32),
    mesh=_MESH,
    scratch_types=[
        pltpu.VMEM((BW * S,), jnp.int32),
        pltpu.VMEM((CHI, 2 * D), jnp.float32),
        pltpu.VMEM((CB, D), jnp.float32),
        pltpu.SemaphoreType.DMA,
    ],
    compiler_params=pltpu.CompilerParams(use_tc_tiling_on_sc=False),
)
def _embed_mean(idx_hbm, table_hbm, out_hbm, idx_v, buf_v, outc_v, sem):
    _body(idx_hbm, table_hbm, out_hbm, idx_v, buf_v, outc_v, sem)


def kernel(inputs, table):
    idx_flat = inputs.astype(jnp.int32).reshape(-1)
    tp = jnp.pad(table, ((0, 0), (0, D)))
    return _embed_mean(idx_flat, tp)
